# baseline (speedup 1.0000x reference)
import jax
import jax.numpy as jnp
from jax import lax
from jax.experimental import pallas as pl
from jax.experimental.pallas import tpu as pltpu
from jax.experimental.pallas import tpu_sc as plsc

N = 10000
D = 128
DH = D // 2
E = 320000
L = 4

LANES = 16
NC = 2
NS = 16
B = 128
ROWS = E // B
RP = 2560
RPW = RP // NS
NP = 10112
NPT = NP // NS


def _seg_body(x2_hbm, srcb_hbm, dst_hbm, ew_hbm, out_hbm,
              src_a, dst_a, ew_a, rows0, rows1,
              acc_sh, sg0, sg1, ss0, ss1):
    cid = lax.axis_index("c")
    sid = lax.axis_index("s")

    pltpu.sync_copy(srcb_hbm.at[cid, pl.ds(sid * RPW, RPW)], src_a)
    pltpu.sync_copy(dst_hbm.at[pl.ds(sid * RPW, RPW)], dst_a)
    pltpu.sync_copy(ew_hbm.at[pl.ds(sid * RPW, RPW)], ew_a)

    zero = jnp.zeros((LANES,), jnp.float32)

    def _zrow(r, c):
        for j in range(DH // LANES):
            rows0[r, pl.ds(LANES * j, LANES)] = zero
        return c

    lax.fori_loop(0, B, _zrow, 0)
    base = sid * NPT
    for k in range(NPT // B):
        pltpu.sync_copy(rows0, acc_sh.at[pl.ds(base + B * k, B)])
    rem = NPT - (NPT // B) * B
    if rem:
        pltpu.sync_copy(rows0.at[pl.ds(0, rem)],
                        acc_sh.at[pl.ds(base + (NPT // B) * B, rem)])
    plsc.subcore_barrier()

    def _compute(buf, t):
        def _grp(gi, cc):
            wv = ew_a[t, pl.ds(LANES * gi, LANES)]
            for rr in range(LANES):
                e = gi * LANES + rr
                w = jnp.full((LANES,), wv[rr], jnp.float32)
                for j in range(DH // LANES):
                    v = buf[e, pl.ds(LANES * j, LANES)]
                    v = jnp.maximum(v, 0.01 * v) * w
                    buf[e, pl.ds(LANES * j, LANES)] = v
            return cc

        lax.fori_loop(0, B // LANES, _grp, 0)

    pltpu.async_copy(x2_hbm.at[src_a.at[0]], rows0, sg0)

    def _iter(t2, c):
        t = 2 * t2

        @pl.when(t2 > 0)
        def _():
            pltpu.make_async_copy(rows1, acc_sh.at[dst_a.at[t - 1]], ss1).wait()

        pltpu.async_copy(x2_hbm.at[src_a.at[t + 1]], rows1, sg1)
        pltpu.make_async_copy(x2_hbm.at[src_a.at[t]], rows0, sg0).wait()
        _compute(rows0, t)
        pltpu.async_copy(rows0, acc_sh.at[dst_a.at[t]], ss0, add=True)
        pltpu.make_async_copy(x2_hbm.at[src_a.at[t + 1]], rows1, sg1).wait()
        _compute(rows1, t + 1)
        pltpu.async_copy(rows1, acc_sh.at[dst_a.at[t + 1]], ss1, add=True)
        pltpu.make_async_copy(rows0, acc_sh.at[dst_a.at[t]], ss0).wait()

        @pl.when(t + 2 < RPW)
        def _():
            pltpu.async_copy(x2_hbm.at[src_a.at[t + 2]], rows0, sg0)

        return c

    lax.fori_loop(0, RPW // 2, _iter, 0)
    pltpu.make_async_copy(rows1, acc_sh.at[dst_a.at[RPW - 1]], ss1).wait()

    plsc.subcore_barrier()
    pltpu.sync_copy(acc_sh.at[pl.ds(base, NPT)],
                    out_hbm.at[cid, pl.ds(base, NPT)])


_seg = pl.kernel(
    _seg_body,
    out_type=jax.ShapeDtypeStruct((NC, NP, DH), jnp.float32),
    mesh=plsc.VectorSubcoreMesh(core_axis_name="c", subcore_axis_name="s",
                                num_cores=NC, num_subcores=NS),
    compiler_params=pltpu.CompilerParams(use_tc_tiling_on_sc=False),
    scratch_types=[
        pltpu.VMEM((RPW, B), jnp.int32),
        pltpu.VMEM((RPW, B), jnp.int32),
        pltpu.VMEM((RPW, B), jnp.float32),
        pltpu.VMEM((B, DH), jnp.float32),
        pltpu.VMEM((B, DH), jnp.float32),
        pltpu.VMEM_SHARED((NP, DH), jnp.float32),
        pltpu.SemaphoreType.DMA,
        pltpu.SemaphoreType.DMA,
        pltpu.SemaphoreType.DMA,
        pltpu.SemaphoreType.DMA,
    ],
)

BN = 1000


def _tc_body(x_ref, p_ref, ws_ref, wn_ref, b_ref, g_ref, o_ref, o2_ref):
    x = x_ref[...]
    h = jnp.maximum(x, 0.01 * x)
    wn = wn_ref[...]
    out = (jnp.dot(h, ws_ref[...], preferred_element_type=jnp.float32)
           + jnp.dot(p_ref[0], wn[0:DH, :], preferred_element_type=jnp.float32)
           + jnp.dot(p_ref[1], wn[DH:D, :], preferred_element_type=jnp.float32)
           + b_ref[...] + g_ref[0] * x)
    o_ref[...] = out
    o2_ref[0] = out[:, 0:DH]
    o2_ref[1] = out[:, DH:D]


_tc = pl.pallas_call(
    _tc_body,
    grid=(N // BN,),
    in_specs=[
        pl.BlockSpec((BN, D), lambda i: (i, 0)),
        pl.BlockSpec((NC, BN, DH), lambda i: (0, i, 0)),
        pl.BlockSpec((D, D), lambda i: (0, 0)),
        pl.BlockSpec((D, D), lambda i: (0, 0)),
        pl.BlockSpec((1, D), lambda i: (0, 0)),
        pl.BlockSpec((1, 1), lambda i: (0, 0)),
    ],
    out_specs=[
        pl.BlockSpec((BN, D), lambda i: (i, 0)),
        pl.BlockSpec((NC, BN, DH), lambda i: (0, i, 0)),
    ],
    out_shape=[
        jax.ShapeDtypeStruct((N, D), jnp.float32),
        jax.ShapeDtypeStruct((NC, N, DH), jnp.float32),
    ],
)


def _pad_rows(a2):
    return jnp.pad(a2, ((0, RP - ROWS), (0, 0)))


def kernel(x, edge_index, edge_weight, W_self, W_neigh, b, gates):
    g = jax.nn.sigmoid(gates)
    srcp = _pad_rows(edge_index[0].reshape(ROWS, B))
    srcb = jnp.stack([srcp, srcp + N])
    dstp = _pad_rows(edge_index[1].reshape(ROWS, B))
    ewp = _pad_rows(edge_weight.reshape(ROWS, B))
    x2 = jnp.concatenate([x[:, 0:DH], x[:, DH:D]], axis=0)
    for i in range(L):
        agg2 = _seg(x2, srcb, dstp, ewp)
        gi = g[i]
        x, x2s = _tc(x, agg2,
                     (1.0 - gi) * W_self[i], (1.0 - gi) * W_neigh[i],
                     ((1.0 - gi) * b[i]).reshape(1, D), gi.reshape(1, 1))
        x2 = x2s.reshape(2 * N, DH)
    return x
